# baseline (speedup 1.0000x reference)
import functools
import numpy as np
import jax
import jax.numpy as jnp
from jax import lax
from jax.experimental import pallas as pl
from jax.experimental.pallas import tpu as pltpu
from jax.experimental.pallas import tpu_sc as plsc

_B, _H, _W, _C = 64, 512, 512, 3

_RECTS = np.array([
    [319, 447, 245, 373, 295, 423, 329, 457], [0, 87, 368, 496, 368, 496, 443, 512],
    [310, 438, 0, 70, 0, 121, 434, 512], [391, 512, 335, 463, 290, 418, 0, 67],
    [131, 259, 74, 202, 23, 151, 269, 397], [183, 311, 347, 475, 197, 325, 11, 139],
    [425, 512, 0, 81, 343, 471, 318, 446], [281, 409, 281, 409, 252, 380, 273, 401],
    [419, 512, 0, 103, 318, 446, 36, 164], [59, 187, 218, 346, 446, 512, 220, 348],
    [0, 109, 297, 425, 250, 378, 325, 453], [97, 225, 118, 246, 34, 162, 411, 512],
    [48, 176, 70, 198, 193, 321, 269, 397], [161, 289, 75, 203, 102, 230, 0, 91],
    [419, 512, 0, 128, 337, 465, 153, 281], [248, 376, 324, 452, 0, 105, 433, 512],
    [407, 512, 274, 402, 356, 484, 223, 351], [351, 479, 133, 261, 423, 512, 0, 78],
    [199, 327, 13, 141, 118, 246, 157, 285], [394, 512, 380, 508, 0, 122, 228, 356],
    [395, 512, 416, 512, 145, 273, 0, 116], [397, 512, 220, 348, 0, 95, 289, 417],
    [0, 70, 117, 245, 447, 512, 375, 503], [100, 228, 276, 404, 68, 196, 120, 248],
    [276, 404, 325, 453, 30, 158, 428, 512], [133, 261, 284, 412, 36, 164, 217, 345],
    [309, 437, 115, 243, 327, 455, 0, 69], [20, 148, 285, 413, 238, 366, 6, 134],
    [13, 141, 2, 130, 170, 298, 104, 232], [187, 315, 15, 143, 413, 512, 177, 305],
    [418, 512, 0, 66, 8, 136, 433, 512], [355, 483, 133, 261, 0, 122, 403, 512],
    [113, 241, 369, 497, 177, 305, 204, 332], [318, 446, 7, 135, 11, 139, 332, 460],
    [86, 214, 118, 246, 32, 160, 237, 365], [436, 512, 389, 512, 62, 190, 0, 124],
    [79, 207, 251, 379, 254, 382, 315, 443], [347, 475, 120, 248, 115, 243, 0, 70],
    [0, 114, 420, 512, 260, 388, 54, 182], [0, 127, 160, 288, 3, 131, 440, 512],
    [290, 418, 370, 498, 312, 440, 106, 234], [361, 489, 220, 348, 0, 87, 416, 512],
    [328, 456, 161, 289, 200, 328, 165, 293], [285, 413, 227, 355, 7, 135, 189, 317],
    [249, 377, 18, 146, 116, 244, 0, 72], [210, 338, 351, 479, 0, 95, 411, 512],
    [251, 379, 0, 108, 66, 194, 196, 324], [81, 209, 362, 490, 324, 452, 0, 122],
    [0, 89, 252, 380, 0, 116, 419, 512], [165, 293, 0, 81, 247, 375, 294, 422],
    [239, 367, 39, 167, 292, 420, 282, 410], [40, 168, 305, 433, 392, 512, 0, 81],
    [234, 362, 386, 512, 403, 512, 247, 375], [341, 469, 437, 512, 431, 512, 238, 366],
    [0, 74, 273, 401, 68, 196, 278, 406], [263, 391, 354, 482, 397, 512, 166, 294],
    [4, 132, 53, 181, 359, 487, 391, 512], [94, 222, 272, 400, 288, 416, 68, 196],
    [218, 346, 334, 462, 122, 250, 79, 207], [274, 402, 240, 368, 0, 115, 0, 120],
    [139, 267, 401, 512, 402, 512, 87, 215], [0, 89, 339, 467, 184, 312, 432, 512],
    [255, 383, 0, 65, 118, 246, 181, 309], [122, 250, 243, 371, 78, 206, 0, 120],
], dtype=np.int32)

_NP = _B * _C
_NW = 32
_PPW = _NP // _NW
_CH = 64
_CPP = _H // _CH
_NCH = _PPW * _CPP
_NBUF = 2


def _sc_cutout_body(x_hbm, o_hbm, buf, *sems):
    sem_in = sems[:_NBUF]
    sem_out = sems[_NBUF:]
    c = lax.axis_index("c")
    s = lax.axis_index("s")
    wid = s * 2 + c
    base = wid * _PPW

    lane = lax.broadcasted_iota(jnp.int32, (16,), 0)

    def chunk_slice(ref, k):
        plane = base + k // _CPP
        r0 = (k % _CPP) * _CH
        return ref.at[plane, pl.ds(r0, _CH), :]

    def table_lookup(col, img_of_w):
        val = jnp.int32(int(_RECTS[img_of_w(0), col]))
        for w in range(1, _NW):
            val = jnp.where(wid == w,
                            jnp.int32(int(_RECTS[img_of_w(w), col])), val)
        return val

    bounds = [[table_lookup(j, lambda w, im=im: 2 * w + im) for j in range(8)]
              for im in range(2)]

    def pick(j, im_sel):
        return jnp.where(im_sel == 0, bounds[0][j], bounds[1][j])

    def mask_chunk(bref, k):
        r0 = (k % _CPP) * _CH
        im_sel = k // (_CPP * _C)
        for rr in range(2):
            y1 = pick(4 * rr + 0, im_sel)
            y2 = pick(4 * rr + 1, im_sel)
            x1 = pick(4 * rr + 2, im_sel)
            x2 = pick(4 * rr + 3, im_sel)
            rlo = jnp.clip(y1 - r0, 0, _CH)
            rhi = jnp.clip(y2 - r0, 0, _CH)
            g1 = x1 // 16
            g2 = (x2 + 15) // 16

            @pl.loop(rlo, rhi)
            def _row(r):
                @pl.loop(g1, g2)
                def _grp(g):
                    cols = g * 16 + lane
                    cm = (cols >= x1) & (cols < x2)
                    vals = bref[r, pl.ds(g * 16, 16)]
                    bref[r, pl.ds(g * 16, 16)] = jnp.where(
                        cm, jnp.float32(0.0), vals)

    pltpu.async_copy(chunk_slice(x_hbm, 0), buf.at[0], sem_in[0])
    pltpu.async_copy(chunk_slice(x_hbm, 1), buf.at[1], sem_in[1])

    @pl.loop(0, _NCH, step=_NBUF)
    def _outer(k2):
        for b in range(_NBUF):
            k = k2 + b
            pltpu.make_async_copy(
                chunk_slice(x_hbm, k), buf.at[b], sem_in[b]).wait()
            mask_chunk(buf.at[b], k)
            pltpu.sync_copy(buf.at[b], chunk_slice(o_hbm, k))

            @pl.when(k + 2 < _NCH)
            def _():
                pltpu.async_copy(
                    chunk_slice(x_hbm, k + 2), buf.at[b], sem_in[b])


def _sc_cutout(x):
    mesh = plsc.VectorSubcoreMesh(core_axis_name="c", subcore_axis_name="s")
    run = pl.kernel(
        _sc_cutout_body,
        out_type=jax.ShapeDtypeStruct((_NP, _H, _W), jnp.float32),
        mesh=mesh,
        scratch_types=(
            [pltpu.VMEM((_NBUF, _CH, _W), jnp.float32)]
            + [pltpu.SemaphoreType.DMA] * (2 * _NBUF)
        ),
    )
    return run(x)


def kernel(inputs):
    x = jnp.transpose(inputs, (0, 3, 1, 2)).reshape(_NP, _H, _W)
    out = _sc_cutout(x)
    return out.reshape(_B, _C, _H, _W).transpose(0, 2, 3, 1)


# speedup vs baseline: 1.0539x; 1.0539x over previous
import functools
import numpy as np
import jax
import jax.numpy as jnp
from jax import lax
from jax.experimental import pallas as pl
from jax.experimental.pallas import tpu as pltpu
from jax.experimental.pallas import tpu_sc as plsc

_B, _H, _W, _C = 64, 512, 512, 3

_RECTS = np.array([
    [319, 447, 245, 373, 295, 423, 329, 457], [0, 87, 368, 496, 368, 496, 443, 512],
    [310, 438, 0, 70, 0, 121, 434, 512], [391, 512, 335, 463, 290, 418, 0, 67],
    [131, 259, 74, 202, 23, 151, 269, 397], [183, 311, 347, 475, 197, 325, 11, 139],
    [425, 512, 0, 81, 343, 471, 318, 446], [281, 409, 281, 409, 252, 380, 273, 401],
    [419, 512, 0, 103, 318, 446, 36, 164], [59, 187, 218, 346, 446, 512, 220, 348],
    [0, 109, 297, 425, 250, 378, 325, 453], [97, 225, 118, 246, 34, 162, 411, 512],
    [48, 176, 70, 198, 193, 321, 269, 397], [161, 289, 75, 203, 102, 230, 0, 91],
    [419, 512, 0, 128, 337, 465, 153, 281], [248, 376, 324, 452, 0, 105, 433, 512],
    [407, 512, 274, 402, 356, 484, 223, 351], [351, 479, 133, 261, 423, 512, 0, 78],
    [199, 327, 13, 141, 118, 246, 157, 285], [394, 512, 380, 508, 0, 122, 228, 356],
    [395, 512, 416, 512, 145, 273, 0, 116], [397, 512, 220, 348, 0, 95, 289, 417],
    [0, 70, 117, 245, 447, 512, 375, 503], [100, 228, 276, 404, 68, 196, 120, 248],
    [276, 404, 325, 453, 30, 158, 428, 512], [133, 261, 284, 412, 36, 164, 217, 345],
    [309, 437, 115, 243, 327, 455, 0, 69], [20, 148, 285, 413, 238, 366, 6, 134],
    [13, 141, 2, 130, 170, 298, 104, 232], [187, 315, 15, 143, 413, 512, 177, 305],
    [418, 512, 0, 66, 8, 136, 433, 512], [355, 483, 133, 261, 0, 122, 403, 512],
    [113, 241, 369, 497, 177, 305, 204, 332], [318, 446, 7, 135, 11, 139, 332, 460],
    [86, 214, 118, 246, 32, 160, 237, 365], [436, 512, 389, 512, 62, 190, 0, 124],
    [79, 207, 251, 379, 254, 382, 315, 443], [347, 475, 120, 248, 115, 243, 0, 70],
    [0, 114, 420, 512, 260, 388, 54, 182], [0, 127, 160, 288, 3, 131, 440, 512],
    [290, 418, 370, 498, 312, 440, 106, 234], [361, 489, 220, 348, 0, 87, 416, 512],
    [328, 456, 161, 289, 200, 328, 165, 293], [285, 413, 227, 355, 7, 135, 189, 317],
    [249, 377, 18, 146, 116, 244, 0, 72], [210, 338, 351, 479, 0, 95, 411, 512],
    [251, 379, 0, 108, 66, 194, 196, 324], [81, 209, 362, 490, 324, 452, 0, 122],
    [0, 89, 252, 380, 0, 116, 419, 512], [165, 293, 0, 81, 247, 375, 294, 422],
    [239, 367, 39, 167, 292, 420, 282, 410], [40, 168, 305, 433, 392, 512, 0, 81],
    [234, 362, 386, 512, 403, 512, 247, 375], [341, 469, 437, 512, 431, 512, 238, 366],
    [0, 74, 273, 401, 68, 196, 278, 406], [263, 391, 354, 482, 397, 512, 166, 294],
    [4, 132, 53, 181, 359, 487, 391, 512], [94, 222, 272, 400, 288, 416, 68, 196],
    [218, 346, 334, 462, 122, 250, 79, 207], [274, 402, 240, 368, 0, 115, 0, 120],
    [139, 267, 401, 512, 402, 512, 87, 215], [0, 89, 339, 467, 184, 312, 432, 512],
    [255, 383, 0, 65, 118, 246, 181, 309], [122, 250, 243, 371, 78, 206, 0, 120],
], dtype=np.int32)

_NP = _B * _C
_NW = 32
_PPW = _NP // _NW
_CH = 32
_CPP = _H // _CH
_NCH = _PPW * _CPP
_NBUF = 6
_AHEAD = _NBUF // 2


def _sc_cutout_body(x_hbm, o_hbm, buf, *sems):
    sem_in = sems[:_NBUF]
    sem_out = sems[_NBUF:]
    c = lax.axis_index("c")
    s = lax.axis_index("s")
    wid = s * 2 + c
    base = wid * _PPW

    lane = lax.broadcasted_iota(jnp.int32, (16,), 0)

    def chunk_slice(ref, k):
        plane = base + k // _CPP
        r0 = (k % _CPP) * _CH
        return ref.at[plane, pl.ds(r0, _CH), :]

    def table_lookup(col, img_of_w):
        val = jnp.int32(int(_RECTS[img_of_w(0), col]))
        for w in range(1, _NW):
            val = jnp.where(wid == w,
                            jnp.int32(int(_RECTS[img_of_w(w), col])), val)
        return val

    bounds = [[table_lookup(j, lambda w, im=im: 2 * w + im) for j in range(8)]
              for im in range(2)]

    def pick(j, im_sel):
        return jnp.where(im_sel == 0, bounds[0][j], bounds[1][j])

    def mask_chunk(bref, k):
        r0 = (k % _CPP) * _CH
        im_sel = k // (_CPP * _C)
        for rr in range(2):
            y1 = pick(4 * rr + 0, im_sel)
            y2 = pick(4 * rr + 1, im_sel)
            x1 = pick(4 * rr + 2, im_sel)
            x2 = pick(4 * rr + 3, im_sel)
            rlo = jnp.clip(y1 - r0, 0, _CH)
            rhi = jnp.clip(y2 - r0, 0, _CH)
            g1 = x1 // 16
            g2 = (x2 + 15) // 16

            @pl.loop(rlo, rhi)
            def _row(r):
                @pl.loop(g1, g2)
                def _grp(g):
                    cols = g * 16 + lane
                    cm = (cols >= x1) & (cols < x2)
                    vals = bref[r, pl.ds(g * 16, 16)]
                    bref[r, pl.ds(g * 16, 16)] = jnp.where(
                        cm, jnp.float32(0.0), vals)

    for j in range(_AHEAD):
        pltpu.async_copy(chunk_slice(x_hbm, j), buf.at[j], sem_in[j])

    @pl.loop(0, _NCH, step=_NBUF)
    def _outer(kb):
        for b in range(_NBUF):
            k = kb + b
            pltpu.make_async_copy(
                chunk_slice(x_hbm, k), buf.at[b], sem_in[b]).wait()
            mask_chunk(buf.at[b], k)
            pltpu.async_copy(buf.at[b], chunk_slice(o_hbm, k), sem_out[b])

            bn = (b + _AHEAD) % _NBUF

            @pl.when(k + _AHEAD < _NCH)
            def _():
                @pl.when(k >= _AHEAD)
                def _():
                    pltpu.make_async_copy(
                        buf.at[bn], chunk_slice(o_hbm, k - _AHEAD),
                        sem_out[bn]).wait()

                pltpu.async_copy(
                    chunk_slice(x_hbm, k + _AHEAD), buf.at[bn], sem_in[bn])

    for b in range(_NBUF):
        pltpu.make_async_copy(
            buf.at[b], chunk_slice(o_hbm, _NCH - _NBUF + b),
            sem_out[b]).wait()


def _sc_cutout(x):
    mesh = plsc.VectorSubcoreMesh(core_axis_name="c", subcore_axis_name="s")
    run = pl.kernel(
        _sc_cutout_body,
        out_type=jax.ShapeDtypeStruct((_NP, _H, _W), jnp.float32),
        mesh=mesh,
        scratch_types=(
            [pltpu.VMEM((_NBUF, _CH, _W), jnp.float32)]
            + [pltpu.SemaphoreType.DMA] * (2 * _NBUF)
        ),
    )
    return run(x)


def kernel(inputs):
    x = jnp.transpose(inputs, (0, 3, 1, 2)).reshape(_NP, _H, _W)
    out = _sc_cutout(x)
    return out.reshape(_B, _C, _H, _W).transpose(0, 2, 3, 1)
